# baseline (device time: 34597 ns/iter reference)
import jax
import jax.numpy as jnp
from jax import lax
from jax.experimental import pallas as pl
from jax.experimental.pallas import tpu as pltpu

N_DEV = 16
PLANE = 4
ZDIM = 4
NA = 512
BP = 2


def kernel(x, w_mat):
    m, k_per = x.shape
    _, n = w_mat.shape
    m_blk = m // N_DEV
    na2 = NA // 2
    nb = n - NA
    nb2 = nb // 2
    sup = m // ZDIM
    piece = sup // BP

    def body(x_ref, w_ref, out_ref, p_ref,
             s1r, s1l, r1r, r1l, s2u, s2d, r2u, r2d,
             b1us, b1ds, b1ur, b1dr, b2rs, b2ls, b2rr, b2lr,
             sem_s1r, sem_s1l, sem_r1r, sem_r1l,
             sem_s2u, sem_s2d, sem_r2u, sem_r2d,
             sem_b1us, sem_b1ds, sem_b1ur, sem_b1dr,
             sem_b2rs, sem_b2ls, sem_b2rr, sem_b2lr):
        my = lax.axis_index("i")
        z = my // PLANE
        p = my % PLANE

        def pz(v):
            return jnp.where(v == 2, 3, jnp.where(v == 3, 2, v))

        zeta = pz(z)
        plane_r = z * PLANE + (p + 1) % PLANE
        plane_l = z * PLANE + (p - 1) % PLANE
        z_up = pz((zeta + 1) % ZDIM) * PLANE + p
        z_dn = pz((zeta - 1) % ZDIM) * PLANE + p

        cols_a_r = pl.ds(0, na2)
        cols_a_l = pl.ds(na2, na2)
        cols_b_u = pl.ds(NA, nb2)
        cols_b_d = pl.ds(NA + nb2, nb2)

        def rc(send, recv, ssem, rsem, idx, dev):
            return pltpu.make_async_remote_copy(
                src_ref=send.at[idx], dst_ref=recv.at[idx],
                send_sem=ssem.at[idx], recv_sem=rsem.at[idx],
                device_id=(dev,), device_id_type=pl.DeviceIdType.MESH,
            )

        def rc2(send, recv, ssem, rsem, i0, i1, dev):
            return pltpu.make_async_remote_copy(
                src_ref=send.at[i0, i1], dst_ref=recv.at[i0, i1],
                send_sem=ssem.at[i0, i1], recv_sem=rsem.at[i0, i1],
                device_id=(dev,), device_id_type=pl.DeviceIdType.MESH,
            )

        barrier_sem = pltpu.get_barrier_semaphore()
        for nbr in (plane_r, plane_l, z_up, z_dn):
            pl.semaphore_signal(
                barrier_sem, inc=1,
                device_id=(nbr,), device_id_type=pl.DeviceIdType.MESH,
            )
        pl.semaphore_wait(barrier_sem, 4)

        p_ref[...] = jnp.dot(
            x_ref[...], w_ref[...], preferred_element_type=jnp.float32
        )

        for s in range(3):
            qr = (p - s - 1) % PLANE
            ql = (p + s + 1) % PLANE
            for k in range(ZDIM):
                zr = pz((zeta - 1 - k) % ZDIM)
                zl = pz((zeta + 1 + k) % ZDIM)
                rows_r = pl.ds((zr * PLANE + qr) * m_blk, m_blk)
                rows_l = pl.ds((zl * PLANE + ql) * m_blk, m_blk)
                if s == 0:
                    s1r[s, zr, :, :] = p_ref[rows_r, cols_a_r]
                    s1l[s, zl, :, :] = p_ref[rows_l, cols_a_l]
                else:
                    rc2(s1r, r1r, sem_s1r, sem_r1r, s - 1, zr, plane_r).wait_recv()
                    rc2(s1l, r1l, sem_s1l, sem_r1l, s - 1, zl, plane_l).wait_recv()
                    s1r[s, zr, :, :] = p_ref[rows_r, cols_a_r] + r1r[s - 1, zr, :, :]
                    s1l[s, zl, :, :] = p_ref[rows_l, cols_a_l] + r1l[s - 1, zl, :, :]
                rc2(s1r, r1r, sem_s1r, sem_r1r, s, zr, plane_r).start()
                rc2(s1l, r1l, sem_s1l, sem_r1l, s, zl, plane_l).start()

            ju_b = pz((zeta - s - 1) % ZDIM)
            jd_b = pz((zeta + s + 1) % ZDIM)
            for h in range(BP):
                rows_bu = pl.ds(ju_b * sup + h * piece, piece)
                rows_bd = pl.ds(jd_b * sup + h * piece, piece)
                if s == 0:
                    b1us[s, h, :, :] = p_ref[rows_bu, cols_b_u]
                    b1ds[s, h, :, :] = p_ref[rows_bd, cols_b_d]
                else:
                    rc2(b1us, b1ur, sem_b1us, sem_b1ur, s - 1, h, z_up).wait_recv()
                    rc2(b1ds, b1dr, sem_b1ds, sem_b1dr, s - 1, h, z_dn).wait_recv()
                    b1us[s, h, :, :] = p_ref[rows_bu, cols_b_u] + b1ur[s - 1, h, :, :]
                    b1ds[s, h, :, :] = p_ref[rows_bd, cols_b_d] + b1dr[s - 1, h, :, :]
                rc2(b1us, b1ur, sem_b1us, sem_b1ur, s, h, z_up).start()
                rc2(b1ds, b1dr, sem_b1ds, sem_b1dr, s, h, z_dn).start()

        for t in range(3):
            ju = pz((zeta - 1 - t) % ZDIM)
            jd = pz((zeta + 1 + t) % ZDIM)
            rc2(s1r, r1r, sem_s1r, sem_r1r, 2, ju, plane_r).wait_recv()
            rc2(s1l, r1l, sem_s1l, sem_r1l, 2, jd, plane_l).wait_recv()
            rows_u = pl.ds((ju * PLANE + p) * m_blk, m_blk)
            rows_d = pl.ds((jd * PLANE + p) * m_blk, m_blk)
            base_u = p_ref[rows_u, cols_a_r] + r1r[2, ju, :, :]
            base_d = p_ref[rows_d, cols_a_l] + r1l[2, jd, :, :]
            if t > 0:
                rc(s2u, r2u, sem_s2u, sem_r2u, t - 1, z_up).wait_recv()
                rc(s2d, r2d, sem_s2d, sem_r2d, t - 1, z_dn).wait_recv()
                base_u = base_u + r2u[t - 1, :, :]
                base_d = base_d + r2d[t - 1, :, :]
            s2u[t, :, :] = base_u
            s2d[t, :, :] = base_d
            rc(s2u, r2u, sem_s2u, sem_r2u, t, z_up).start()
            rc(s2d, r2d, sem_s2d, sem_r2d, t, z_dn).start()

            if t == 0:
                for h in range(BP):
                    rc2(b1us, b1ur, sem_b1us, sem_b1ur, 2, h, z_up).wait_recv()
                    rc2(b1ds, b1dr, sem_b1ds, sem_b1dr, 2, h, z_dn).wait_recv()
            qbr = (p - t - 1) % PLANE
            qbl = (p + t + 1) % PLANE
            rows_br = pl.ds((z * PLANE + qbr) * m_blk, m_blk)
            rows_bl = pl.ds((z * PLANE + qbl) * m_blk, m_blk)
            hqr = qbr // BP
            oqr = (qbr % BP) * m_blk
            hql = qbl // BP
            oql = (qbl % BP) * m_blk
            base_br = (p_ref[rows_br, cols_b_u]
                       + b1ur[2, hqr, pl.ds(oqr, m_blk), :])
            base_bl = (p_ref[rows_bl, cols_b_d]
                       + b1dr[2, hql, pl.ds(oql, m_blk), :])
            if t > 0:
                rc(b2rs, b2rr, sem_b2rs, sem_b2rr, t - 1, plane_r).wait_recv()
                rc(b2ls, b2lr, sem_b2ls, sem_b2lr, t - 1, plane_l).wait_recv()
                base_br = base_br + b2rr[t - 1, :, :]
                base_bl = base_bl + b2lr[t - 1, :, :]
            b2rs[t, :, :] = base_br
            b2ls[t, :, :] = base_bl
            rc(b2rs, b2rr, sem_b2rs, sem_b2rr, t, plane_r).start()
            rc(b2ls, b2lr, sem_b2ls, sem_b2lr, t, plane_l).start()

        rc2(s1r, r1r, sem_s1r, sem_r1r, 2, z, plane_r).wait_recv()
        rc2(s1l, r1l, sem_s1l, sem_r1l, 2, z, plane_l).wait_recv()
        rc(s2u, r2u, sem_s2u, sem_r2u, 2, z_up).wait_recv()
        rc(s2d, r2d, sem_s2d, sem_r2d, 2, z_dn).wait_recv()
        rc(b2rs, b2rr, sem_b2rs, sem_b2rr, 2, plane_r).wait_recv()
        rc(b2ls, b2lr, sem_b2ls, sem_b2lr, 2, plane_l).wait_recv()
        rows_m = pl.ds(my * m_blk, m_blk)
        hp = p // BP
        op = (p % BP) * m_blk
        yar = p_ref[rows_m, cols_a_r] + r1r[2, z, :, :] + r2u[2, :, :]
        yal = p_ref[rows_m, cols_a_l] + r1l[2, z, :, :] + r2d[2, :, :]
        ybr = p_ref[rows_m, cols_b_u] + b1ur[2, hp, pl.ds(op, m_blk), :] + b2rr[2, :, :]
        ybl = p_ref[rows_m, cols_b_d] + b1dr[2, hp, pl.ds(op, m_blk), :] + b2lr[2, :, :]
        out_ref[:, cols_a_r] = yar * (1.0 / (1.0 + jnp.exp(-yar)))
        out_ref[:, cols_a_l] = yal * (1.0 / (1.0 + jnp.exp(-yal)))
        out_ref[:, cols_b_u] = ybr * (1.0 / (1.0 + jnp.exp(-ybr)))
        out_ref[:, cols_b_d] = ybl * (1.0 / (1.0 + jnp.exp(-ybl)))

        for s in range(3):
            for zz in range(ZDIM):
                rc2(s1r, r1r, sem_s1r, sem_r1r, s, zz, plane_r).wait_send()
                rc2(s1l, r1l, sem_s1l, sem_r1l, s, zz, plane_l).wait_send()
            for h in range(BP):
                rc2(b1us, b1ur, sem_b1us, sem_b1ur, s, h, z_up).wait_send()
                rc2(b1ds, b1dr, sem_b1ds, sem_b1dr, s, h, z_dn).wait_send()
            rc(s2u, r2u, sem_s2u, sem_r2u, s, z_up).wait_send()
            rc(s2d, r2d, sem_s2d, sem_r2d, s, z_dn).wait_send()
            rc(b2rs, b2rr, sem_b2rs, sem_b2rr, s, plane_r).wait_send()
            rc(b2ls, b2lr, sem_b2ls, sem_b2lr, s, plane_l).wait_send()

    out_shape = jax.ShapeDtypeStruct((m_blk, n), jnp.float32)
    dma = pltpu.SemaphoreType.DMA
    return pl.pallas_call(
        body,
        out_shape=out_shape,
        in_specs=[
            pl.BlockSpec(memory_space=pltpu.VMEM),
            pl.BlockSpec(memory_space=pltpu.VMEM),
        ],
        out_specs=pl.BlockSpec(memory_space=pltpu.VMEM),
        scratch_shapes=[
            pltpu.VMEM((m, n), jnp.float32),
            pltpu.VMEM((3, ZDIM, m_blk, na2), jnp.float32),
            pltpu.VMEM((3, ZDIM, m_blk, na2), jnp.float32),
            pltpu.VMEM((3, ZDIM, m_blk, na2), jnp.float32),
            pltpu.VMEM((3, ZDIM, m_blk, na2), jnp.float32),
            pltpu.VMEM((3, m_blk, na2), jnp.float32),
            pltpu.VMEM((3, m_blk, na2), jnp.float32),
            pltpu.VMEM((3, m_blk, na2), jnp.float32),
            pltpu.VMEM((3, m_blk, na2), jnp.float32),
            pltpu.VMEM((3, BP, piece, nb2), jnp.float32),
            pltpu.VMEM((3, BP, piece, nb2), jnp.float32),
            pltpu.VMEM((3, BP, piece, nb2), jnp.float32),
            pltpu.VMEM((3, BP, piece, nb2), jnp.float32),
            pltpu.VMEM((3, m_blk, nb2), jnp.float32),
            pltpu.VMEM((3, m_blk, nb2), jnp.float32),
            pltpu.VMEM((3, m_blk, nb2), jnp.float32),
            pltpu.VMEM((3, m_blk, nb2), jnp.float32),
            dma((3, ZDIM)), dma((3, ZDIM)), dma((3, ZDIM)), dma((3, ZDIM)),
            dma((3,)), dma((3,)), dma((3,)), dma((3,)),
            dma((3, BP)), dma((3, BP)), dma((3, BP)), dma((3, BP)),
            dma((3,)), dma((3,)), dma((3,)), dma((3,)),
        ],
        compiler_params=pltpu.CompilerParams(collective_id=0),
    )(x, w_mat)


# device time: 33446 ns/iter; 1.0344x vs baseline; 1.0344x over previous
import jax
import jax.numpy as jnp
from jax import lax
from jax.experimental import pallas as pl
from jax.experimental.pallas import tpu as pltpu

N_DEV = 16
PLANE = 4
ZDIM = 4
NA = 512
BP = 2


def kernel(x, w_mat):
    m, k_per = x.shape
    _, n = w_mat.shape
    m_blk = m // N_DEV
    na2 = NA // 2
    nb = n - NA
    nb2 = nb // 2
    sup = m // ZDIM
    piece = sup // BP

    def body(x_ref, w_ref, out_ref, p_ref,
             s1r, s1l, r1r, r1l, b1us, b1ds, b1ur, b1dr,
             a2s, a2r, b2s, b2r,
             sem_s1r, sem_s1l, sem_r1r, sem_r1l,
             sem_b1us, sem_b1ds, sem_b1ur, sem_b1dr,
             sem_a2s, sem_a2r, sem_b2s, sem_b2r):
        my = lax.axis_index("i")
        z = my // PLANE
        p = my % PLANE

        def pz(v):
            return jnp.where(v == 2, 3, jnp.where(v == 3, 2, v))

        zeta = pz(z)
        plane_r = z * PLANE + (p + 1) % PLANE
        plane_l = z * PLANE + (p - 1) % PLANE
        z_up = pz((zeta + 1) % ZDIM) * PLANE + p
        z_dn = pz((zeta - 1) % ZDIM) * PLANE + p

        cols_a_r = pl.ds(0, na2)
        cols_a_l = pl.ds(na2, na2)
        cols_a = pl.ds(0, NA)
        cols_b_u = pl.ds(NA, nb2)
        cols_b_d = pl.ds(NA + nb2, nb2)
        cols_b = pl.ds(NA, nb)

        def rc(send, recv, ssem, rsem, idx, dev):
            return pltpu.make_async_remote_copy(
                src_ref=send.at[idx], dst_ref=recv.at[idx],
                send_sem=ssem.at[idx], recv_sem=rsem.at[idx],
                device_id=(dev,), device_id_type=pl.DeviceIdType.MESH,
            )

        def rc2(send, recv, ssem, rsem, i0, i1, dev):
            return pltpu.make_async_remote_copy(
                src_ref=send.at[i0, i1], dst_ref=recv.at[i0, i1],
                send_sem=ssem.at[i0, i1], recv_sem=rsem.at[i0, i1],
                device_id=(dev,), device_id_type=pl.DeviceIdType.MESH,
            )

        def rc_from_p(rows, cols, recv, ssem, rsem, i0, i1, dev):
            return pltpu.make_async_remote_copy(
                src_ref=p_ref.at[rows, cols], dst_ref=recv.at[i0, i1],
                send_sem=ssem.at[i0, i1], recv_sem=rsem.at[i0, i1],
                device_id=(dev,), device_id_type=pl.DeviceIdType.MESH,
            )

        p_ref[...] = jnp.dot(
            x_ref[...], w_ref[...], preferred_element_type=jnp.float32
        )

        plane_diag = z * PLANE + (p + 2) % PLANE
        z_far = pz((zeta + 2) % ZDIM) * PLANE + p
        barrier_sem = pltpu.get_barrier_semaphore()
        for nbr in (plane_r, plane_l, plane_diag, z_up, z_dn, z_far):
            pl.semaphore_signal(
                barrier_sem, inc=1,
                device_id=(nbr,), device_id_type=pl.DeviceIdType.MESH,
            )
        pl.semaphore_wait(barrier_sem, 6)

        for s in range(3):
            qr = (p - s - 1) % PLANE
            ql = (p + s + 1) % PLANE
            for k in range(ZDIM):
                zk = (z + 1 + k) % ZDIM
                rows_r = pl.ds((zk * PLANE + qr) * m_blk, m_blk)
                rows_l = pl.ds((zk * PLANE + ql) * m_blk, m_blk)
                if s == 0:
                    rc_from_p(rows_r, cols_a_r, r1r, sem_s1r, sem_r1r,
                              s, zk, plane_r).start()
                    rc_from_p(rows_l, cols_a_l, r1l, sem_s1l, sem_r1l,
                              s, zk, plane_l).start()
                else:
                    rc2(s1r, r1r, sem_s1r, sem_r1r, s - 1, zk, plane_r).wait_recv()
                    rc2(s1l, r1l, sem_s1l, sem_r1l, s - 1, zk, plane_l).wait_recv()
                    s1r[s, zk, :, :] = p_ref[rows_r, cols_a_r] + r1r[s - 1, zk, :, :]
                    s1l[s, zk, :, :] = p_ref[rows_l, cols_a_l] + r1l[s - 1, zk, :, :]
                    rc2(s1r, r1r, sem_s1r, sem_r1r, s, zk, plane_r).start()
                    rc2(s1l, r1l, sem_s1l, sem_r1l, s, zk, plane_l).start()

            ju_b = pz((zeta - s - 1) % ZDIM)
            jd_b = pz((zeta + s + 1) % ZDIM)
            for h in range(BP):
                rows_bu = pl.ds(ju_b * sup + h * piece, piece)
                rows_bd = pl.ds(jd_b * sup + h * piece, piece)
                if s == 0:
                    rc_from_p(rows_bu, cols_b_u, b1ur, sem_b1us, sem_b1ur,
                              s, h, z_up).start()
                    rc_from_p(rows_bd, cols_b_d, b1dr, sem_b1ds, sem_b1dr,
                              s, h, z_dn).start()
                else:
                    rc2(b1us, b1ur, sem_b1us, sem_b1ur, s - 1, h, z_up).wait_recv()
                    rc2(b1ds, b1dr, sem_b1ds, sem_b1dr, s - 1, h, z_dn).wait_recv()
                    b1us[s, h, :, :] = p_ref[rows_bu, cols_b_u] + b1ur[s - 1, h, :, :]
                    b1ds[s, h, :, :] = p_ref[rows_bd, cols_b_d] + b1dr[s - 1, h, :, :]
                    rc2(b1us, b1ur, sem_b1us, sem_b1ur, s, h, z_up).start()
                    rc2(b1ds, b1dr, sem_b1ds, sem_b1dr, s, h, z_dn).start()

        for h in range(BP):
            rc2(b1us, b1ur, sem_b1us, sem_b1ur, 2, h, z_up).wait_recv()
            rc2(b1ds, b1dr, sem_b1ds, sem_b1dr, 2, h, z_dn).wait_recv()
        for delta in range(1, PLANE):
            q = (p + delta) % PLANE
            tgt = z * PLANE + q
            rows_q = pl.ds((z * PLANE + q) * m_blk, m_blk)
            hq = q // BP
            oq = (q % BP) * m_blk
            b2s[delta - 1, :, 0:nb2] = (
                p_ref[rows_q, cols_b_u] + b1ur[2, hq, pl.ds(oq, m_blk), :]
            )
            b2s[delta - 1, :, nb2:nb] = (
                p_ref[rows_q, cols_b_d] + b1dr[2, hq, pl.ds(oq, m_blk), :]
            )
            rc(b2s, b2r, sem_b2s, sem_b2r, delta - 1, tgt).start()

        for delta in range(1, ZDIM):
            j = (z + delta) % ZDIM
            tgt = j * PLANE + p
            rc2(s1r, r1r, sem_s1r, sem_r1r, 2, j, plane_r).wait_recv()
            rc2(s1l, r1l, sem_s1l, sem_r1l, 2, j, plane_l).wait_recv()
            rows_j = pl.ds((j * PLANE + p) * m_blk, m_blk)
            a2s[delta - 1, :, 0:na2] = p_ref[rows_j, cols_a_r] + r1r[2, j, :, :]
            a2s[delta - 1, :, na2:NA] = p_ref[rows_j, cols_a_l] + r1l[2, j, :, :]
            rc(a2s, a2r, sem_a2s, sem_a2r, delta - 1, tgt).start()

        rc2(s1r, r1r, sem_s1r, sem_r1r, 2, z, plane_r).wait_recv()
        rc2(s1l, r1l, sem_s1l, sem_r1l, 2, z, plane_l).wait_recv()
        for d in range(3):
            rc(a2s, a2r, sem_a2s, sem_a2r, d, my).wait_recv()
            rc(b2s, b2r, sem_b2s, sem_b2r, d, my).wait_recv()
        rows_m = pl.ds(my * m_blk, m_blk)
        hp = p // BP
        op = (p % BP) * m_blk
        yar = (p_ref[rows_m, cols_a_r] + r1r[2, z, :, :]
               + a2r[0, :, 0:na2] + a2r[1, :, 0:na2] + a2r[2, :, 0:na2])
        yal = (p_ref[rows_m, cols_a_l] + r1l[2, z, :, :]
               + a2r[0, :, na2:NA] + a2r[1, :, na2:NA] + a2r[2, :, na2:NA])
        ybu = (p_ref[rows_m, cols_b_u] + b1ur[2, hp, pl.ds(op, m_blk), :]
               + b2r[0, :, 0:nb2] + b2r[1, :, 0:nb2] + b2r[2, :, 0:nb2])
        ybd = (p_ref[rows_m, cols_b_d] + b1dr[2, hp, pl.ds(op, m_blk), :]
               + b2r[0, :, nb2:nb] + b2r[1, :, nb2:nb] + b2r[2, :, nb2:nb])
        out_ref[:, cols_a_r] = yar * (1.0 / (1.0 + jnp.exp(-yar)))
        out_ref[:, cols_a_l] = yal * (1.0 / (1.0 + jnp.exp(-yal)))
        out_ref[:, cols_b_u] = ybu * (1.0 / (1.0 + jnp.exp(-ybu)))
        out_ref[:, cols_b_d] = ybd * (1.0 / (1.0 + jnp.exp(-ybd)))

        for s in range(3):
            for zz in range(ZDIM):
                rc2(s1r, r1r, sem_s1r, sem_r1r, s, zz, plane_r).wait_send()
                rc2(s1l, r1l, sem_s1l, sem_r1l, s, zz, plane_l).wait_send()
            for h in range(BP):
                rc2(b1us, b1ur, sem_b1us, sem_b1ur, s, h, z_up).wait_send()
                rc2(b1ds, b1dr, sem_b1ds, sem_b1dr, s, h, z_dn).wait_send()
            rc(a2s, a2r, sem_a2s, sem_a2r, s, my).wait_send()
            rc(b2s, b2r, sem_b2s, sem_b2r, s, my).wait_send()

    out_shape = jax.ShapeDtypeStruct((m_blk, n), jnp.float32)
    dma = pltpu.SemaphoreType.DMA
    return pl.pallas_call(
        body,
        out_shape=out_shape,
        in_specs=[
            pl.BlockSpec(memory_space=pltpu.VMEM),
            pl.BlockSpec(memory_space=pltpu.VMEM),
        ],
        out_specs=pl.BlockSpec(memory_space=pltpu.VMEM),
        scratch_shapes=[
            pltpu.VMEM((m, n), jnp.float32),
            pltpu.VMEM((3, ZDIM, m_blk, na2), jnp.float32),
            pltpu.VMEM((3, ZDIM, m_blk, na2), jnp.float32),
            pltpu.VMEM((3, ZDIM, m_blk, na2), jnp.float32),
            pltpu.VMEM((3, ZDIM, m_blk, na2), jnp.float32),
            pltpu.VMEM((3, BP, piece, nb2), jnp.float32),
            pltpu.VMEM((3, BP, piece, nb2), jnp.float32),
            pltpu.VMEM((3, BP, piece, nb2), jnp.float32),
            pltpu.VMEM((3, BP, piece, nb2), jnp.float32),
            pltpu.VMEM((3, m_blk, NA), jnp.float32),
            pltpu.VMEM((3, m_blk, NA), jnp.float32),
            pltpu.VMEM((3, m_blk, nb), jnp.float32),
            pltpu.VMEM((3, m_blk, nb), jnp.float32),
            dma((3, ZDIM)), dma((3, ZDIM)), dma((3, ZDIM)), dma((3, ZDIM)),
            dma((3, BP)), dma((3, BP)), dma((3, BP)), dma((3, BP)),
            dma((3,)), dma((3,)), dma((3,)), dma((3,)),
        ],
        compiler_params=pltpu.CompilerParams(collective_id=0),
    )(x, w_mat)


# device time: 31970 ns/iter; 1.0822x vs baseline; 1.0462x over previous
import jax
import jax.numpy as jnp
from jax import lax
from jax.experimental import pallas as pl
from jax.experimental.pallas import tpu as pltpu

N_DEV = 16
PLANE = 4
ZDIM = 4
NA = 512
BP = 2


def kernel(x, w_mat):
    m, k_per = x.shape
    _, n = w_mat.shape
    m_blk = m // N_DEV
    na2 = NA // 2
    nb = n - NA
    nb2 = nb // 2
    sup = m // ZDIM
    piece = sup // BP

    def body(x_ref, w_ref, out_ref, p_ref,
             xr1s, xr1r, xr2s, xr2r, yr1s, yr1r, yr2s, yr2r,
             b1us, b1ds, b1ur, b1dr, a2s, a2r, b2s, b2r,
             sem_xr1s, sem_xr1r, sem_xr2s, sem_xr2r,
             sem_yr1s, sem_yr1r, sem_yr2s, sem_yr2r,
             sem_b1us, sem_b1ds, sem_b1ur, sem_b1dr,
             sem_a2s, sem_a2r, sem_b2s, sem_b2r):
        my = lax.axis_index("i")
        z = my // PLANE
        p = my % PLANE

        def pz(v):
            return jnp.where(v == 2, 3, jnp.where(v == 3, 2, v))

        zeta = pz(z)
        xpart = p + 1 - 2 * (p % 2)
        ypart = 3 - p
        diagp = 3 - xpart
        dev_x = z * PLANE + xpart
        dev_y = z * PLANE + ypart
        dev_diag = z * PLANE + diagp
        z_up = pz((zeta + 1) % ZDIM) * PLANE + p
        z_dn = pz((zeta - 1) % ZDIM) * PLANE + p
        z_far = pz((zeta + 2) % ZDIM) * PLANE + p

        cs_a = pl.ds(0, na2)
        cs_b = pl.ds(na2, na2)
        cols_b_u = pl.ds(NA, nb2)
        cols_b_d = pl.ds(NA + nb2, nb2)

        def rc(send, recv, ssem, rsem, idx, dev):
            return pltpu.make_async_remote_copy(
                src_ref=send.at[idx], dst_ref=recv.at[idx],
                send_sem=ssem.at[idx], recv_sem=rsem.at[idx],
                device_id=(dev,), device_id_type=pl.DeviceIdType.MESH,
            )

        def rc2(send, recv, ssem, rsem, i0, i1, dev):
            return pltpu.make_async_remote_copy(
                src_ref=send.at[i0, i1], dst_ref=recv.at[i0, i1],
                send_sem=ssem.at[i0, i1], recv_sem=rsem.at[i0, i1],
                device_id=(dev,), device_id_type=pl.DeviceIdType.MESH,
            )

        def rc_plain(send, recv, ssem, rsem, dev):
            return pltpu.make_async_remote_copy(
                src_ref=send, dst_ref=recv, send_sem=ssem, recv_sem=rsem,
                device_id=(dev,), device_id_type=pl.DeviceIdType.MESH,
            )

        def rc_from_p(rows, cols, recv, ssem, rsem, i0, i1, dev):
            return pltpu.make_async_remote_copy(
                src_ref=p_ref.at[rows, cols], dst_ref=recv.at[i0, i1],
                send_sem=ssem.at[i0, i1], recv_sem=rsem.at[i0, i1],
                device_id=(dev,), device_id_type=pl.DeviceIdType.MESH,
            )

        p_ref[...] = jnp.dot(
            x_ref[...], w_ref[...], preferred_element_type=jnp.float32
        )

        barrier_sem = pltpu.get_barrier_semaphore()
        for nbr in (dev_x, dev_y, dev_diag, z_up, z_dn, z_far):
            pl.semaphore_signal(
                barrier_sem, inc=1,
                device_id=(nbr,), device_id_type=pl.DeviceIdType.MESH,
            )
        pl.semaphore_wait(barrier_sem, 6)

        for slot, q in ((0, diagp), (1, xpart)):
            for j in range(ZDIM):
                rows = pl.ds((j * PLANE + q) * m_blk, m_blk)
                xr1s[slot, j, :, :] = p_ref[rows, cs_a]
            rc(xr1s, xr1r, sem_xr1s, sem_xr1r, slot, dev_x).start()
        for slot, q in ((0, diagp), (1, ypart)):
            for j in range(ZDIM):
                rows = pl.ds((j * PLANE + q) * m_blk, m_blk)
                yr1s[slot, j, :, :] = p_ref[rows, cs_b]
            rc(yr1s, yr1r, sem_yr1s, sem_yr1r, slot, dev_y).start()

        ju_b = pz((zeta - 1) % ZDIM)
        jd_b = pz((zeta + 1) % ZDIM)
        for h in range(BP):
            rc_from_p(pl.ds(ju_b * sup + h * piece, piece), cols_b_u,
                      b1ur, sem_b1us, sem_b1ur, 0, h, z_up).start()
            rc_from_p(pl.ds(jd_b * sup + h * piece, piece), cols_b_d,
                      b1dr, sem_b1ds, sem_b1dr, 0, h, z_dn).start()

        rc(xr1s, xr1r, sem_xr1s, sem_xr1r, 0, dev_x).wait_recv()
        for j in range(ZDIM):
            rows = pl.ds((j * PLANE + ypart) * m_blk, m_blk)
            xr2s[j, :, :] = p_ref[rows, cs_a] + xr1r[0, j, :, :]
        rc_plain(xr2s, xr2r, sem_xr2s, sem_xr2r, dev_y).start()
        rc(yr1s, yr1r, sem_yr1s, sem_yr1r, 0, dev_y).wait_recv()
        for j in range(ZDIM):
            rows = pl.ds((j * PLANE + xpart) * m_blk, m_blk)
            yr2s[j, :, :] = p_ref[rows, cs_b] + yr1r[0, j, :, :]
        rc_plain(yr2s, yr2r, sem_yr2s, sem_yr2r, dev_x).start()

        for s in range(1, 3):
            ju_b = pz((zeta - s - 1) % ZDIM)
            jd_b = pz((zeta + s + 1) % ZDIM)
            for h in range(BP):
                rows_bu = pl.ds(ju_b * sup + h * piece, piece)
                rows_bd = pl.ds(jd_b * sup + h * piece, piece)
                rc2(b1us, b1ur, sem_b1us, sem_b1ur, s - 1, h, z_up).wait_recv()
                rc2(b1ds, b1dr, sem_b1ds, sem_b1dr, s - 1, h, z_dn).wait_recv()
                b1us[s, h, :, :] = p_ref[rows_bu, cols_b_u] + b1ur[s - 1, h, :, :]
                b1ds[s, h, :, :] = p_ref[rows_bd, cols_b_d] + b1dr[s - 1, h, :, :]
                rc2(b1us, b1ur, sem_b1us, sem_b1ur, s, h, z_up).start()
                rc2(b1ds, b1dr, sem_b1ds, sem_b1dr, s, h, z_dn).start()

        rc(xr1s, xr1r, sem_xr1s, sem_xr1r, 1, dev_x).wait_recv()
        rc(yr1s, yr1r, sem_yr1s, sem_yr1r, 1, dev_y).wait_recv()
        rc_plain(xr2s, xr2r, sem_xr2s, sem_xr2r, dev_y).wait_recv()
        rc_plain(yr2s, yr2r, sem_yr2s, sem_yr2r, dev_x).wait_recv()
        for delta in range(1, ZDIM):
            j = (z + delta) % ZDIM
            tgt = j * PLANE + p
            rows_j = pl.ds((j * PLANE + p) * m_blk, m_blk)
            a2s[delta - 1, :, 0:na2] = (
                p_ref[rows_j, cs_a] + xr1r[1, j, :, :] + xr2r[j, :, :])
            a2s[delta - 1, :, na2:NA] = (
                p_ref[rows_j, cs_b] + yr1r[1, j, :, :] + yr2r[j, :, :])
            rc(a2s, a2r, sem_a2s, sem_a2r, delta - 1, tgt).start()

        for h in range(BP):
            rc2(b1us, b1ur, sem_b1us, sem_b1ur, 2, h, z_up).wait_recv()
            rc2(b1ds, b1dr, sem_b1ds, sem_b1dr, 2, h, z_dn).wait_recv()
        for delta in range(1, PLANE):
            q = (p + delta) % PLANE
            tgt = z * PLANE + q
            rows_q = pl.ds((z * PLANE + q) * m_blk, m_blk)
            hq = q // BP
            oq = (q % BP) * m_blk
            b2s[delta - 1, :, 0:nb2] = (
                p_ref[rows_q, cols_b_u] + b1ur[2, hq, pl.ds(oq, m_blk), :])
            b2s[delta - 1, :, nb2:nb] = (
                p_ref[rows_q, cols_b_d] + b1dr[2, hq, pl.ds(oq, m_blk), :])
            rc(b2s, b2r, sem_b2s, sem_b2r, delta - 1, tgt).start()

        for d in range(3):
            rc(a2s, a2r, sem_a2s, sem_a2r, d, my).wait_recv()
            rc(b2s, b2r, sem_b2s, sem_b2r, d, my).wait_recv()
        rows_m = pl.ds(my * m_blk, m_blk)
        hp = p // BP
        op = (p % BP) * m_blk
        yar = (p_ref[rows_m, cs_a] + xr1r[1, z, :, :] + xr2r[z, :, :]
               + a2r[0, :, 0:na2] + a2r[1, :, 0:na2] + a2r[2, :, 0:na2])
        yal = (p_ref[rows_m, cs_b] + yr1r[1, z, :, :] + yr2r[z, :, :]
               + a2r[0, :, na2:NA] + a2r[1, :, na2:NA] + a2r[2, :, na2:NA])
        ybu = (p_ref[rows_m, cols_b_u] + b1ur[2, hp, pl.ds(op, m_blk), :]
               + b2r[0, :, 0:nb2] + b2r[1, :, 0:nb2] + b2r[2, :, 0:nb2])
        ybd = (p_ref[rows_m, cols_b_d] + b1dr[2, hp, pl.ds(op, m_blk), :]
               + b2r[0, :, nb2:nb] + b2r[1, :, nb2:nb] + b2r[2, :, nb2:nb])
        out_ref[:, cs_a] = yar * (1.0 / (1.0 + jnp.exp(-yar)))
        out_ref[:, cs_b] = yal * (1.0 / (1.0 + jnp.exp(-yal)))
        out_ref[:, cols_b_u] = ybu * (1.0 / (1.0 + jnp.exp(-ybu)))
        out_ref[:, cols_b_d] = ybd * (1.0 / (1.0 + jnp.exp(-ybd)))

        for slot in range(2):
            rc(xr1s, xr1r, sem_xr1s, sem_xr1r, slot, dev_x).wait_send()
            rc(yr1s, yr1r, sem_yr1s, sem_yr1r, slot, dev_y).wait_send()
        rc_plain(xr2s, xr2r, sem_xr2s, sem_xr2r, dev_y).wait_send()
        rc_plain(yr2s, yr2r, sem_yr2s, sem_yr2r, dev_x).wait_send()
        for s in range(3):
            for h in range(BP):
                rc2(b1us, b1ur, sem_b1us, sem_b1ur, s, h, z_up).wait_send()
                rc2(b1ds, b1dr, sem_b1ds, sem_b1dr, s, h, z_dn).wait_send()
            rc(a2s, a2r, sem_a2s, sem_a2r, s, my).wait_send()
            rc(b2s, b2r, sem_b2s, sem_b2r, s, my).wait_send()

    out_shape = jax.ShapeDtypeStruct((m_blk, n), jnp.float32)
    dma = pltpu.SemaphoreType.DMA
    return pl.pallas_call(
        body,
        out_shape=out_shape,
        in_specs=[
            pl.BlockSpec(memory_space=pltpu.VMEM),
            pl.BlockSpec(memory_space=pltpu.VMEM),
        ],
        out_specs=pl.BlockSpec(memory_space=pltpu.VMEM),
        scratch_shapes=[
            pltpu.VMEM((m, n), jnp.float32),
            pltpu.VMEM((2, ZDIM, m_blk, na2), jnp.float32),
            pltpu.VMEM((2, ZDIM, m_blk, na2), jnp.float32),
            pltpu.VMEM((ZDIM, m_blk, na2), jnp.float32),
            pltpu.VMEM((ZDIM, m_blk, na2), jnp.float32),
            pltpu.VMEM((2, ZDIM, m_blk, na2), jnp.float32),
            pltpu.VMEM((2, ZDIM, m_blk, na2), jnp.float32),
            pltpu.VMEM((ZDIM, m_blk, na2), jnp.float32),
            pltpu.VMEM((ZDIM, m_blk, na2), jnp.float32),
            pltpu.VMEM((3, BP, piece, nb2), jnp.float32),
            pltpu.VMEM((3, BP, piece, nb2), jnp.float32),
            pltpu.VMEM((3, BP, piece, nb2), jnp.float32),
            pltpu.VMEM((3, BP, piece, nb2), jnp.float32),
            pltpu.VMEM((3, m_blk, NA), jnp.float32),
            pltpu.VMEM((3, m_blk, NA), jnp.float32),
            pltpu.VMEM((3, m_blk, nb), jnp.float32),
            pltpu.VMEM((3, m_blk, nb), jnp.float32),
            dma((2,)), dma((2,)), dma, dma,
            dma((2,)), dma((2,)), dma, dma,
            dma((3, BP)), dma((3, BP)), dma((3, BP)), dma((3, BP)),
            dma((3,)), dma((3,)), dma((3,)), dma((3,)),
        ],
        compiler_params=pltpu.CompilerParams(collective_id=0),
    )(x, w_mat)
